# baseline (speedup 1.0000x reference)
import functools

import jax
import jax.numpy as jnp
from jax.experimental import pallas as pl

F32 = jnp.float32
NE, ED = 1024, 64


def _mm_bias_relu_body(m_ref, w_ref, b_ref, o_ref):
    y = jnp.dot(m_ref[...], w_ref[...], preferred_element_type=F32)
    o_ref[...] = jnp.maximum(y + b_ref[...], 0.0)


def _conv_s2_body(xp_ref, w_ref, b_ref, o_ref, *, n, so, cin, cout):
    rows = n * so * so * so
    acc = jnp.zeros((rows, cout), F32)
    for kd in range(4):
        for kh in range(4):
            for kw in range(4):
                p = ((kd % 2) * 2 + (kh % 2)) * 2 + (kw % 2)
                od, oh, ow = kd // 2, kh // 2, kw // 2
                sl = xp_ref[p * n:(p + 1) * n,
                            od:od + so, oh:oh + so, ow:ow + so, :]
                sl = sl.reshape(rows, cin)
                t = (kd * 4 + kh) * 4 + kw
                acc = acc + jnp.dot(sl, w_ref[t], preferred_element_type=F32)
    o_ref[...] = jnp.maximum(acc + b_ref[...], 0.0)


def _tap27(xp_ref, w_ref, n, so, cin, cout):
    rows = n * so * so * so
    acc = jnp.zeros((rows, cout), F32)
    for dd in range(3):
        for dh in range(3):
            for dw in range(3):
                sl = xp_ref[:, dd:dd + so, dh:dh + so, dw:dw + so, :]
                sl = sl.reshape(rows, cin)
                t = (dd * 3 + dh) * 3 + dw
                acc = acc + jnp.dot(sl, w_ref[t], preferred_element_type=F32)
    return acc


def _conv3_vq_body(xp_ref, w_ref, b_ref, cbt_ref, cb_ref,
                   zq_ref, loss_ref, *, so, total_rows):
    z = _tap27(xp_ref, w_ref, 1, so, ED, ED) + b_ref[...]
    zcb = jnp.dot(z, cbt_ref[...], preferred_element_type=F32)
    z2 = jnp.sum(z * z, axis=1, keepdims=True)
    cbt = cbt_ref[...]
    c2 = jnp.sum(cbt * cbt, axis=0, keepdims=True)
    d = (z2 + c2) - 2.0 * zcb
    m = jnp.min(d, axis=1, keepdims=True)
    iota = jax.lax.broadcasted_iota(jnp.int32, d.shape, 1)
    idx = jnp.min(jnp.where(d == m, iota, NE), axis=1, keepdims=True)
    onehot = (iota == idx).astype(F32)
    zq = jnp.dot(onehot, cb_ref[...], preferred_element_type=F32)
    diff = zq - z
    s = jnp.sum(diff * diff, axis=1, keepdims=True)
    part = jnp.sum(s, axis=0, keepdims=True) * (1.0 / (total_rows * ED))

    @pl.when(pl.program_id(0) == 0)
    def _init():
        loss_ref[...] = jnp.zeros((1, 1), F32)

    loss_ref[...] += part
    zq_ref[...] = zq


def _dec1_body(xp_ref, w_ref, b_ref, o_ref, *, n, so):
    acc = _tap27(xp_ref, w_ref, n, so, ED, 256)
    o_ref[...] = jnp.maximum(acc + b_ref[...], 0.0)


def _dec2_body(xp_ref, w_ref, b_ref, o_ref, *, n, so):
    acc = _tap27(xp_ref, w_ref, n, so, 32, 8)
    o_ref[...] = acc + b_ref[...]


def _pad1(x):
    return jnp.pad(x, ((0, 0), (1, 1), (1, 1), (1, 1), (0, 0)))


def _parity_split(xp):
    parts = [xp[:, pd::2, ph::2, pw::2, :]
             for pd in range(2) for ph in range(2) for pw in range(2)]
    return jnp.concatenate(parts, axis=0)


def kernel(patched_tsdf, enc_w1, enc_b1, enc_w2, enc_b2, enc_w3, enc_b3,
           codebook, dec_w1, dec_b1, dec_w2, dec_b2):
    B = patched_tsdf.shape[0]

    w1r = jnp.transpose(enc_w1[:, 0], (1, 2, 3, 0)).reshape(64, 32)
    w2r = jnp.transpose(enc_w2, (2, 3, 4, 1, 0)).reshape(64, 32, 64)
    w3r = jnp.transpose(enc_w3, (2, 3, 4, 1, 0)).reshape(27, ED, ED)
    cbt = codebook.T

    def _subpixel_w(w, cin, cout_):
        blocks = []
        for dd in (-1, 0, 1):
            for dh in (-1, 0, 1):
                for dw in (-1, 0, 1):
                    cols = []
                    for pd in range(2):
                        for ph in range(2):
                            for pw in range(2):
                                kd = 2 * dd + 2 - pd
                                kh = 2 * dh + 2 - ph
                                kw = 2 * dw + 2 - pw
                                if all(0 <= k < 4 for k in (kd, kh, kw)):
                                    cols.append(jnp.transpose(w[:, :, kd, kh, kw]))
                                else:
                                    cols.append(jnp.zeros((cin, cout_), F32))
                    blocks.append(jnp.concatenate(cols, axis=1))
        return jnp.stack(blocks, axis=0)

    w4r = _subpixel_w(dec_w1, ED, 32)
    w5r = _subpixel_w(dec_w2, 32, 1)
    b4r = jnp.tile(dec_b1, 8)[None, :]
    b5r = jnp.tile(dec_b2, 8)[None, :]

    x = patched_tsdf[:, 0]
    xpad = jnp.pad(x, ((0, 0), (1, 1), (1, 1), (1, 1)))
    taps = [xpad[:, kd:kd + 32:2, kh:kh + 32:2, kw:kw + 32:2]
            for kd in range(4) for kh in range(4) for kw in range(4)]
    m1 = jnp.stack(taps, axis=-1).reshape(B * 16 ** 3, 64)
    h1 = pl.pallas_call(
        _mm_bias_relu_body,
        out_shape=jax.ShapeDtypeStruct((B * 16 ** 3, 32), F32),
    )(m1, w1r, enc_b1[None, :])

    h1p = _parity_split(_pad1(h1.reshape(B, 16, 16, 16, 32)))
    h2 = pl.pallas_call(
        functools.partial(_conv_s2_body, n=B, so=8, cin=32, cout=64),
        out_shape=jax.ShapeDtypeStruct((B * 8 ** 3, 64), F32),
    )(h1p, w2r, enc_b2[None, :])

    h2p = _pad1(h2.reshape(B, 8, 8, 8, 64))
    zq, loss = pl.pallas_call(
        functools.partial(_conv3_vq_body, so=8, total_rows=B * 8 ** 3),
        grid=(B,),
        in_specs=[
            pl.BlockSpec((1, 10, 10, 10, 64), lambda i: (i, 0, 0, 0, 0)),
            pl.BlockSpec((27, ED, ED), lambda i: (0, 0, 0)),
            pl.BlockSpec((1, ED), lambda i: (0, 0)),
            pl.BlockSpec((ED, NE), lambda i: (0, 0)),
            pl.BlockSpec((NE, ED), lambda i: (0, 0)),
        ],
        out_specs=(pl.BlockSpec((8 ** 3, ED), lambda i: (i, 0)),
                   pl.BlockSpec((1, 1), lambda i: (0, 0))),
        out_shape=(jax.ShapeDtypeStruct((B * 8 ** 3, ED), F32),
                   jax.ShapeDtypeStruct((1, 1), F32)),
    )(h2p, w3r, enc_b3[None, :], cbt, codebook)
    loss = loss[0, 0]

    zqp = _pad1(zq.reshape(B, 8, 8, 8, ED))
    g = pl.pallas_call(
        functools.partial(_dec1_body, n=B, so=8),
        out_shape=jax.ShapeDtypeStruct((B * 8 ** 3, 256), F32),
    )(zqp, w4r, b4r)
    g3 = g.reshape(B, 8, 8, 8, 2, 2, 2, 32)
    g3 = jnp.transpose(g3, (0, 1, 4, 2, 5, 3, 6, 7)).reshape(B, 16, 16, 16, 32)

    gp = _pad1(g3)
    xh = pl.pallas_call(
        functools.partial(_dec2_body, n=1, so=16),
        grid=(B,),
        in_specs=[
            pl.BlockSpec((1, 18, 18, 18, 32), lambda i: (i, 0, 0, 0, 0)),
            pl.BlockSpec((27, 32, 8), lambda i: (0, 0, 0)),
            pl.BlockSpec((1, 8), lambda i: (0, 0)),
        ],
        out_specs=pl.BlockSpec((16 ** 3, 8), lambda i: (i, 0)),
        out_shape=jax.ShapeDtypeStruct((B * 16 ** 3, 8), F32),
    )(gp, w5r, b5r)
    xh = xh.reshape(B, 16, 16, 16, 2, 2, 2)
    xh = jnp.transpose(xh, (0, 1, 4, 2, 5, 3, 6)).reshape(B, 1, 32, 32, 32)

    return (xh, loss, loss)


# speedup vs baseline: 14.1938x; 14.1938x over previous
import functools

import jax
import jax.numpy as jnp
import numpy as np
from jax import lax
from jax.experimental import pallas as pl
from jax.experimental.pallas import tpu as pltpu

F32 = jnp.float32
NE, ED = 1024, 64


def _enc_vq_body(x_ref, b1_ref, w1_ref, w2_ref, b2_ref, w3_ref, b3_ref,
                 cbt_ref, cb_ref, zq_ref, loss_ref,
                 xs, xp, h1s, h1p, h2s):
    B = x_ref.shape[0]
    rows1 = B * 16 * 16
    rows2 = B * 8 ** 3

    xs[...] = jnp.zeros(xs.shape, F32)
    xs[:, 1:33, 1:33, 1:33] = x_ref[...]
    for pd in range(2):
        for ph in range(2):
            p = pd * 2 + ph
            xp[p * B:(p + 1) * B] = xs[:, pd:34:2, ph:34:2, :]

    acc1 = jnp.zeros((rows1, 512), F32)
    for kd in range(4):
        for kh in range(4):
            p = (kd % 2) * 2 + kh % 2
            jd, jh = kd // 2, kh // 2
            sl = xp[p * B:(p + 1) * B, jd:jd + 16, jh:jh + 16, :]
            acc1 = acc1 + jnp.dot(sl.reshape(rows1, 34), w1_ref[kd * 4 + kh],
                                  preferred_element_type=F32)
    h1 = jnp.maximum(acc1 + b1_ref[...], 0.0)
    h1s[...] = h1.reshape(B, 16, 16, 16, 32)

    h1p[...] = jnp.zeros(h1p.shape, F32)
    for pd in range(2):
        for ph in range(2):
            for pw in range(2):
                t = (1 - pd) * 4 + (1 - ph) * 2 + (1 - pw)
                h1p[t * B:(t + 1) * B, pd:pd + 8, ph:ph + 8, pw:pw + 8, :] = (
                    h1s[:, pd:16:2, ph:16:2, pw:16:2, :])

    def c2_body(t, acc):
        kd, kh = t // 4, t % 4
        for kw in range(4):
            p = (kd % 2) * 4 + (kh % 2) * 2 + (kw % 2)
            sl = h1p[pl.ds(p * B, B), pl.ds(kd // 2, 8), pl.ds(kh // 2, 8),
                     kw // 2:kw // 2 + 8, :]
            acc = acc + jnp.dot(sl.reshape(rows2, 32), w2_ref[t * 4 + kw],
                                preferred_element_type=F32)
        return acc
    acc2 = lax.fori_loop(0, 16, c2_body, jnp.zeros((rows2, 64), F32))
    h2 = jnp.maximum(acc2 + b2_ref[...], 0.0)
    h2s[...] = jnp.zeros(h2s.shape, F32)
    h2s[:, 1:9, 1:9, 1:9, :] = h2.reshape(B, 8, 8, 8, 64)

    def c3_body(t, acc):
        dd, dh = t // 3, t % 3
        for dw in range(3):
            sl = h2s[:, pl.ds(dd, 8), pl.ds(dh, 8), dw:dw + 8, :]
            acc = acc + jnp.dot(sl.reshape(rows2, ED), w3_ref[t * 3 + dw],
                                preferred_element_type=F32)
        return acc
    acc3 = lax.fori_loop(0, 9, c3_body, jnp.zeros((rows2, ED), F32))
    z = acc3 + b3_ref[...]

    cbt = cbt_ref[...]
    c2s = jnp.sum(cbt * cbt, axis=0, keepdims=True)
    chunk = 512
    loss_acc = jnp.zeros((1, 1), F32)
    for c0 in range(0, rows2, chunk):
        zc = z[c0:c0 + chunk]
        zcb = jnp.dot(zc, cbt, preferred_element_type=F32)
        z2 = jnp.sum(zc * zc, axis=1, keepdims=True)
        d = (z2 + c2s) - 2.0 * zcb
        m = jnp.min(d, axis=1, keepdims=True)
        iota = lax.broadcasted_iota(jnp.int32, d.shape, 1)
        idx = jnp.min(jnp.where(d == m, iota, NE), axis=1, keepdims=True)
        onehot = (iota == idx).astype(F32)
        zqc = jnp.dot(onehot, cb_ref[...], preferred_element_type=F32)
        diff = zqc - zc
        s = jnp.sum(diff * diff, axis=1, keepdims=True)
        loss_acc = loss_acc + jnp.sum(s, axis=0, keepdims=True)
        zq_ref[c0:c0 + chunk, :] = zqc
    loss_ref[...] = loss_acc * (1.0 / (rows2 * ED))


def _dec_body(zq_ref, w4_ref, b4_ref, w5_ref, b5_ref, out_ref, zqs, gs):
    B = zqs.shape[0]
    rows2 = B * 8 ** 3

    zqs[...] = jnp.zeros(zqs.shape, F32)
    zqs[:, 1:9, 1:9, 1:9, :] = zq_ref[...].reshape(B, 8, 8, 8, ED)

    def d1_body(t, acc):
        dd, dh = t // 3, t % 3
        for dw in range(3):
            sl = zqs[:, pl.ds(dd, 8), pl.ds(dh, 8), dw:dw + 8, :]
            acc = acc + jnp.dot(sl.reshape(rows2, ED), w4_ref[t * 3 + dw],
                                preferred_element_type=F32)
        return acc
    acc4 = lax.fori_loop(0, 9, d1_body, jnp.zeros((rows2, 256), F32))
    g = jnp.maximum(acc4 + b4_ref[...], 0.0)
    gs[...] = jnp.zeros(gs.shape, F32)
    gs[:, 1:9, 1:9, 1:9, :] = g.reshape(B, 8, 8, 8, 256)

    def d2_body(t, acc):
        dd, dh = t // 3, t % 3
        for dw in range(3):
            sl = gs[:, pl.ds(dd, 8), pl.ds(dh, 8), dw:dw + 8, :]
            acc = acc + jnp.dot(sl.reshape(rows2, 256), w5_ref[t * 3 + dw],
                                preferred_element_type=F32)
        return acc
    acc5 = lax.fori_loop(0, 9, d2_body, jnp.zeros((rows2, 64), F32))
    out_ref[...] = acc5 + b5_ref[...]


def _sel_conv1(w1):
    S = np.zeros((4, 34, 16), np.float32)
    for kw in range(4):
        for ow in range(16):
            S[kw, 2 * ow + kw, ow] = 1.0
    w1p = w1[:, 0]
    b = jnp.einsum('kwo,cdek->dewoc', jnp.asarray(S), w1p)
    return b.reshape(16, 34, 512)


_D1 = np.zeros((3, 2, 4), np.float32)
for _a in range(3):
    for _p in range(2):
        _k = 2 * _a - _p
        if 0 <= _k < 4:
            _D1[_a, _p, _k] = 1.0

_E2 = np.zeros((3, 2, 4, 4), np.float32)
for _s in range(4):
    _u, _q = _s // 2, _s % 2
    for _d in (-1, 0, 1):
        _k = 2 * _d + 2 - _q
        if 0 <= _k < 4:
            _E2[(_u + _d) // 2 + 1, (_u + _d) % 2, _s, _k] = 1.0


def kernel(patched_tsdf, enc_w1, enc_b1, enc_w2, enc_b2, enc_w3, enc_b3,
           codebook, dec_w1, dec_b1, dec_w2, dec_b2):
    B = patched_tsdf.shape[0]

    w1r = _sel_conv1(enc_w1)
    b1r = jnp.tile(enc_b1, 16)[None, :]
    w2r = jnp.transpose(enc_w2, (2, 3, 4, 1, 0)).reshape(64, 32, 64)
    w3r = jnp.transpose(enc_w3, (2, 3, 4, 1, 0)).reshape(27, ED, ED)
    cbt = codebook.T
    d1 = jnp.asarray(_D1)
    w4r = jnp.einsum('apk,bql,crm,oiklm->abcipqro', d1, d1, d1,
                     dec_w1).reshape(27, ED, 256)
    b4r = jnp.tile(dec_b1, 8)[None, :]
    e2 = jnp.asarray(_E2)
    w5r = jnp.einsum('apsk,bqtl,crum,xklm->abcpqrxstu', e2, e2, e2,
                     dec_w2[0]).reshape(27, 256, 64)
    b5r = jnp.broadcast_to(dec_b2[None, :], (1, 64))

    zq, loss = pl.pallas_call(
        _enc_vq_body,
        out_shape=(jax.ShapeDtypeStruct((B * 8 ** 3, ED), F32),
                   jax.ShapeDtypeStruct((1, 1), F32)),
        scratch_shapes=[
            pltpu.VMEM((B, 34, 34, 34), F32),
            pltpu.VMEM((4 * B, 17, 17, 34), F32),
            pltpu.VMEM((B, 16, 16, 16, 32), F32),
            pltpu.VMEM((8 * B, 9, 9, 9, 32), F32),
            pltpu.VMEM((B, 10, 10, 10, 64), F32),
        ],
    )(patched_tsdf.reshape(B, 32, 32, 32), b1r, w1r, w2r,
      enc_b2[None, :], w3r, enc_b3[None, :], cbt, codebook)

    out5 = pl.pallas_call(
        _dec_body,
        out_shape=jax.ShapeDtypeStruct((B * 8 ** 3, 64), F32),
        scratch_shapes=[
            pltpu.VMEM((B, 10, 10, 10, 64), F32),
            pltpu.VMEM((B, 10, 10, 10, 256), F32),
        ],
    )(zq, w4r, b4r, w5r, b5r)

    loss = loss[0, 0]
    xh = out5.reshape(B, 8, 8, 8, 4, 4, 4)
    xh = jnp.transpose(xh, (0, 1, 4, 2, 5, 3, 6)).reshape(B, 1, 32, 32, 32)
    return (xh, loss, loss)
